# Initial kernel scaffold; baseline (speedup 1.0000x reference)
import jax
import jax.numpy as jnp
from jax.experimental import pallas as pl


def kernel(H, hyperedges, rv, W1, b1, W2, b2):
    raise NotImplementedError("write your pallas kernel here")


# speedup vs baseline: 46.9693x; 46.9693x over previous
import functools

import jax
import jax.numpy as jnp
from jax import lax
from jax.experimental import pallas as pl
from jax.experimental.pallas import tpu as pltpu
from jax.experimental.pallas import tpu_sc as plsc

V = 10000
DIN = 128
D1 = 16
NCLS = 40
D2 = 48
K = 4
W0 = 1.0 / 5.0

HE = 80000
NW = 32
WIN = 128
NWIN = 20
EW = WIN * NWIN
HEP = NW * EW
NPAD = HEP - HE

VR = 10240
VE = 12288
DUMMY_MASK = 2047

_HIGH = lax.Precision.HIGHEST


def _tc1_body(h_ref, w1_ref, rv_ref, q_ref, hw1_ref):
    H = h_ref[...]
    rv = rv_ref[...]
    q_ref[...] = jnp.sum(H * rv[None, :], axis=1)
    hw1_ref[...] = lax.dot_general(H, w1_ref[...], (((1,), (0,)), ((), ())),
                                   precision=_HIGH)


def _tc1(h_pad, w1, rv):
    return pl.pallas_call(
        _tc1_body,
        out_shape=[jax.ShapeDtypeStruct((VR,), jnp.float32),
                   jax.ShapeDtypeStruct((VR, D1), jnp.float32)],
    )(h_pad, w1, rv)


def _tc2_body(degp_ref, hw1_ref, ds_ref, dinv_ref, g1_ref):
    deg = degp_ref[0, :] + degp_ref[1, :] + 1.0
    ds = lax.rsqrt(deg)
    dinv = 1.0 / deg
    ds_ref[...] = ds
    dinv_ref[...] = dinv
    g1_ref[0:VR, :] = W0 * ds[:, None] * hw1_ref[...]
    g1_ref[VR:VE, :] = jnp.zeros((VE - VR, D1), jnp.float32)


def _tc2(deg_part, hw1):
    return pl.pallas_call(
        _tc2_body,
        out_shape=[jax.ShapeDtypeStruct((VR,), jnp.float32),
                   jax.ShapeDtypeStruct((VR,), jnp.float32),
                   jax.ShapeDtypeStruct((VE, D1), jnp.float32)],
    )(deg_part, hw1)


def _tc3_body(t1_ref, hw1_ref, ds_ref, dinv_ref, b1_ref, w2_ref,
              hw2_ref, g2_ref):
    T = t1_ref[0] + t1_ref[1]
    ds = ds_ref[...]
    dinv = dinv_ref[...]
    pre = ds[:, None] * T + dinv[:, None] * hw1_ref[...] + b1_ref[...][None, :]
    h = jnp.maximum(pre, 0.0)
    rowid = lax.broadcasted_iota(jnp.int32, (VR, D1), 0)
    h = jnp.where(rowid < V, h, 0.0)
    hw2 = lax.dot_general(h, w2_ref[...], (((1,), (0,)), ((), ())),
                          precision=_HIGH)
    hw2_ref[...] = hw2
    g2_ref[0:VR, :] = W0 * ds[:, None] * hw2
    g2_ref[VR:VE, :] = jnp.zeros((VE - VR, D2), jnp.float32)


def _tc3(t1_part, hw1, ds, dinv, b1, w2p):
    return pl.pallas_call(
        _tc3_body,
        out_shape=[jax.ShapeDtypeStruct((VR, D2), jnp.float32),
                   jax.ShapeDtypeStruct((VE, D2), jnp.float32)],
    )(t1_part, hw1, ds, dinv, b1, w2p)


def _tc4_body(t2_ref, hw2_ref, ds_ref, dinv_ref, b2_ref, out_ref):
    T = t2_ref[0] + t2_ref[1]
    pre = (ds_ref[...][:, None] * T + dinv_ref[...][:, None] * hw2_ref[...]
           + b2_ref[...][None, :])
    o = jnp.maximum(pre, 0.0)
    logits = o[0:V, 0:NCLS]
    m = jnp.max(logits, axis=1, keepdims=True)
    e = jnp.exp(logits - m)
    lse = jnp.log(jnp.sum(e, axis=1, keepdims=True)) + m
    out_ref[...] = logits - lse


def _tc4(t2_part, hw2, ds, dinv, b2p):
    return pl.pallas_call(
        _tc4_body,
        out_shape=jax.ShapeDtypeStruct((V, NCLS), jnp.float32),
    )(t2_part, hw2, ds, dinv, b2p)


_MESH = plsc.VectorSubcoreMesh(core_axis_name="c", subcore_axis_name="s")
_SC_PARAMS = pltpu.CompilerParams(needs_layout_passes=False,
                                  use_tc_tiling_on_sc=False)


@functools.partial(
    pl.kernel,
    out_type=[jax.ShapeDtypeStruct((14, HEP), jnp.int32),
              jax.ShapeDtypeStruct((2, VR), jnp.float32)],
    mesh=_MESH,
    compiler_params=_SC_PARAMS,
    scratch_types=[
        pltpu.VMEM((VR,), jnp.float32),
        pltpu.VMEM((K * EW,), jnp.int32),
        pltpu.VMEM((K, NWIN, WIN), jnp.int32),
        pltpu.VMEM((14 * EW,), jnp.int32),
        pltpu.VMEM((K * EW,), jnp.float32),
        pltpu.VMEM_SHARED((VR,), jnp.float32),
    ],
)
def _sc_edges(hed2_hbm, q_hbm, z1_hbm, src_out, deg_out,
              q_v, vf, vb, of, dgf, deg_sh):
    c = lax.axis_index("c")
    s = lax.axis_index("s")
    wid = s * 2 + c
    base = wid * EW

    pltpu.sync_copy(z1_hbm.at[pl.ds(s * 640, 640)],
                    deg_sh.at[pl.ds(s * 640, 640)])
    plsc.subcore_barrier()

    for j in range(K):
        pltpu.sync_copy(hed2_hbm.at[j, pl.ds(base, EW)],
                        vf.at[pl.ds(j * EW, EW)])
        for w in range(NWIN):
            pltpu.sync_copy(hed2_hbm.at[j, pl.ds(base + w * WIN, WIN)],
                            vb.at[j, w])
    pltpu.sync_copy(q_hbm, q_v)

    lane = lax.iota(jnp.int32, 16)

    @pl.loop(0, EW // 16)
    def _grp(g):
        off = g * 16
        sl = pl.ds(off, 16)
        v = [vf[pl.ds(j * EW + off, 16)] for j in range(K)]
        p = [plsc.load_gather(q_v, [v[j]]) for j in range(K)]
        pmax = jnp.maximum(jnp.maximum(p[0], p[1]), jnp.maximum(p[2], p[3]))
        pmin = jnp.minimum(jnp.minimum(p[0], p[1]), jnp.minimum(p[2], p[3]))
        isx = [p[j] == pmax for j in range(K)]
        isn = [p[j] == pmin for j in range(K)]
        imax = [isx[0],
                isx[1] & ~isx[0],
                isx[2] & ~(isx[0] | isx[1]),
                isx[3] & ~(isx[0] | isx[1] | isx[2])]
        imin = [isn[0],
                isn[1] & ~isn[0],
                isn[2] & ~(isn[0] | isn[1]),
                isn[3] & ~(isn[0] | isn[1] | isn[2])]
        se = jnp.where(imax[0], v[0],
                       jnp.where(imax[1], v[1],
                                 jnp.where(imax[2], v[2], v[3])))
        ie = jnp.where(imin[0], v[0],
                       jnp.where(imin[1], v[1],
                                 jnp.where(imin[2], v[2], v[3])))
        med = [(v[j] != se) & (v[j] != ie) for j in range(K)]
        medf = [med[j].astype(jnp.float32) for j in range(K)]
        nmed = medf[0] + medf[1] + medf[2] + medf[3]
        dummy = V + ((off + wid * 16 + lane) & DUMMY_MASK)
        of[sl] = se
        of[pl.ds(EW + off, 16)] = ie
        for j in range(K):
            of[pl.ds((2 + j) * EW + off, 16)] = jnp.where(med[j], se, dummy)
            of[pl.ds((6 + j) * EW + off, 16)] = jnp.where(med[j], ie, dummy)
            of[pl.ds((10 + j) * EW + off, 16)] = jnp.where(med[j], v[j], dummy)
            extf = imax[j].astype(jnp.float32) + imin[j].astype(jnp.float32)
            dgf[pl.ds(j * EW + off, 16)] = W0 * (2.0 * medf[j]
                                                 + extf * (1.0 + nmed))

    for r in range(14):
        pltpu.sync_copy(of.at[pl.ds(r * EW, EW)],
                        src_out.at[r, pl.ds(base, EW)])
    for j in range(K):
        for w in range(NWIN):
            pltpu.sync_copy(dgf.at[pl.ds(j * EW + w * WIN, WIN)],
                            deg_sh.at[vb.at[j, w]], add=True)

    plsc.subcore_barrier()
    pltpu.sync_copy(deg_sh.at[pl.ds(s * 640, 640)],
                    deg_out.at[c, pl.ds(s * 640, 640)])


_PAIRS = [(0, 1), (1, 0)]
for _j in range(K):
    _PAIRS += [(2 + _j, 14 + _j), (10 + _j, 0), (10 + _j, 1), (6 + _j, 14 + _j)]


def _make_scatter(d):

    @functools.partial(
        pl.kernel,
        out_type=jax.ShapeDtypeStruct((2, VR, d), jnp.float32),
        mesh=_MESH,
        compiler_params=_SC_PARAMS,
        scratch_types=[
            pltpu.VMEM((18, NWIN, WIN), jnp.int32),
            pltpu.VMEM((WIN, d), jnp.float32),
            pltpu.VMEM((WIN, d), jnp.float32),
            pltpu.VMEM_SHARED((VR, d), jnp.float32),
            pltpu.SemaphoreType.DMA,
            pltpu.SemaphoreType.DMA,
        ],
    )
    def _scatter(g_hbm, src_hbm, hed_hbm, z_hbm, t_out, idx, r0, r1, t_sh,
                 sem0, sem1):
        c = lax.axis_index("c")
        s = lax.axis_index("s")
        wid = s * 2 + c

        pltpu.sync_copy(z_hbm.at[pl.ds(s * 640, 640), :],
                        t_sh.at[pl.ds(s * 640, 640), :])
        plsc.subcore_barrier()

        pltpu.sync_copy(src_hbm.at[:, wid], idx.at[pl.ds(0, 14)])
        pltpu.sync_copy(hed_hbm.at[:, wid], idx.at[pl.ds(14, K)])

        for (sr, dr) in _PAIRS:
            pltpu.async_copy(g_hbm.at[idx.at[sr, 0]], r0, sem0)

            @pl.loop(0, NWIN // 2)
            def _win(i):
                wa = i * 2
                wb = wa + 1
                pltpu.make_async_copy(g_hbm.at[idx.at[sr, wa]], r0,
                                      sem0).wait()
                pltpu.async_copy(g_hbm.at[idx.at[sr, wb]], r1, sem1)
                pltpu.sync_copy(r0, t_sh.at[idx.at[dr, wa]], add=True)
                pltpu.make_async_copy(g_hbm.at[idx.at[sr, wb]], r1,
                                      sem1).wait()

                @pl.when(wb + 1 < NWIN)
                def _():
                    pltpu.async_copy(g_hbm.at[idx.at[sr, wb + 1]], r0, sem0)

                pltpu.sync_copy(r1, t_sh.at[idx.at[dr, wb]], add=True)

        plsc.subcore_barrier()
        pltpu.sync_copy(t_sh.at[pl.ds(s * 640, 640), :],
                        t_out.at[c, pl.ds(s * 640, 640), :])

    return _scatter


_scatter_d1 = _make_scatter(D1)
_scatter_d2 = _make_scatter(D2)


def kernel(H, hyperedges, rv, W1, b1, W2, b2):
    f32 = jnp.float32
    h_pad = jnp.zeros((VR, DIN), f32).at[0:V, :].set(H)
    padv = (V + (jnp.arange(NPAD, dtype=jnp.int32) % (VR - V)))[:, None]
    he_pad = jnp.concatenate(
        [hyperedges.astype(jnp.int32), jnp.broadcast_to(padv, (NPAD, K))], 0)
    hed2 = he_pad.T
    hed4 = hed2.reshape(K, NW, NWIN, WIN)
    w2p = jnp.zeros((D1, D2), f32).at[:, 0:NCLS].set(W2)
    b2p = jnp.zeros((D2,), f32).at[0:NCLS].set(b2)
    z1 = jnp.zeros((VR,), f32)
    zd1 = jnp.zeros((VR, D1), f32)
    zd2 = jnp.zeros((VR, D2), f32)

    q, hw1 = _tc1(h_pad, W1, rv)
    src_all, deg_part = _sc_edges(hed2, q, z1)
    src4 = src_all.reshape(14, NW, NWIN, WIN)
    ds, dinv, g1 = _tc2(deg_part, hw1)
    t1 = _scatter_d1(g1, src4, hed4, zd1)
    hw2, g2 = _tc3(t1, hw1, ds, dinv, b1, w2p)
    t2 = _scatter_d2(g2, src4, hed4, zd2)
    return _tc4(t2, hw2, ds, dinv, b2p)


# speedup vs baseline: 90.2354x; 1.9212x over previous
import functools

import jax
import jax.numpy as jnp
from jax import lax
from jax.experimental import pallas as pl
from jax.experimental.pallas import tpu as pltpu
from jax.experimental.pallas import tpu_sc as plsc

V = 10000
DIN = 128
D1 = 16
NCLS = 40
D2 = 48
K = 4
W0 = 1.0 / 5.0

HE = 80000
NW = 32
WIN = 128
NWIN = 20
EW = WIN * NWIN
HEP = NW * EW
NPAD = HEP - HE

VR = 10240
VE = 12288
DUMMY_MASK = 2047

_HIGH = lax.Precision.HIGHEST


def _tc1_body(h_ref, w1_ref, rv_ref, q_ref, hw1_ref):
    H = h_ref[...]
    rv = rv_ref[...]
    q_ref[...] = jnp.sum(H * rv[None, :], axis=1)
    hw1_ref[...] = lax.dot_general(H, w1_ref[...], (((1,), (0,)), ((), ())),
                                   precision=_HIGH)


def _tc1(h_pad, w1, rv):
    return pl.pallas_call(
        _tc1_body,
        out_shape=[jax.ShapeDtypeStruct((VR,), jnp.float32),
                   jax.ShapeDtypeStruct((VR, D1), jnp.float32)],
    )(h_pad, w1, rv)


def _tc2_body(degp_ref, hw1_ref, ds_ref, dinv_ref, g1_ref):
    deg = degp_ref[0, :] + degp_ref[1, :] + 1.0
    ds = lax.rsqrt(deg)
    dinv = 1.0 / deg
    ds_ref[...] = ds
    dinv_ref[...] = dinv
    g1_ref[0:VR, :] = W0 * ds[:, None] * hw1_ref[...]
    g1_ref[VR:VE, :] = jnp.zeros((VE - VR, D1), jnp.float32)


def _tc2(deg_part, hw1):
    return pl.pallas_call(
        _tc2_body,
        out_shape=[jax.ShapeDtypeStruct((VR,), jnp.float32),
                   jax.ShapeDtypeStruct((VR,), jnp.float32),
                   jax.ShapeDtypeStruct((VE, D1), jnp.float32)],
    )(deg_part, hw1)


def _tc3_body(t1_ref, hw1_ref, ds_ref, dinv_ref, b1_ref, w2_ref,
              hw2_ref, g2_ref):
    T = t1_ref[0] + t1_ref[1]
    ds = ds_ref[...]
    dinv = dinv_ref[...]
    pre = ds[:, None] * T + dinv[:, None] * hw1_ref[...] + b1_ref[...][None, :]
    h = jnp.maximum(pre, 0.0)
    rowid = lax.broadcasted_iota(jnp.int32, (VR, D1), 0)
    h = jnp.where(rowid < V, h, 0.0)
    hw2 = lax.dot_general(h, w2_ref[...], (((1,), (0,)), ((), ())),
                          precision=_HIGH)
    hw2_ref[...] = hw2
    g2_ref[0:VR, :] = W0 * ds[:, None] * hw2
    g2_ref[VR:VE, :] = jnp.zeros((VE - VR, D2), jnp.float32)


def _tc3(t1_part, hw1, ds, dinv, b1, w2p):
    return pl.pallas_call(
        _tc3_body,
        out_shape=[jax.ShapeDtypeStruct((VR, D2), jnp.float32),
                   jax.ShapeDtypeStruct((VE, D2), jnp.float32)],
    )(t1_part, hw1, ds, dinv, b1, w2p)


def _tc4_body(t2_ref, hw2_ref, ds_ref, dinv_ref, b2_ref, out_ref):
    T = t2_ref[0] + t2_ref[1]
    pre = (ds_ref[...][:, None] * T + dinv_ref[...][:, None] * hw2_ref[...]
           + b2_ref[...][None, :])
    o = jnp.maximum(pre, 0.0)
    logits = o[0:V, 0:NCLS]
    m = jnp.max(logits, axis=1, keepdims=True)
    e = jnp.exp(logits - m)
    lse = jnp.log(jnp.sum(e, axis=1, keepdims=True)) + m
    out_ref[...] = logits - lse


def _tc4(t2_part, hw2, ds, dinv, b2p):
    return pl.pallas_call(
        _tc4_body,
        out_shape=jax.ShapeDtypeStruct((V, NCLS), jnp.float32),
    )(t2_part, hw2, ds, dinv, b2p)


_MESH = plsc.VectorSubcoreMesh(core_axis_name="c", subcore_axis_name="s")
_SC_PARAMS = pltpu.CompilerParams(needs_layout_passes=False,
                                  use_tc_tiling_on_sc=False)


@functools.partial(
    pl.kernel,
    out_type=[jax.ShapeDtypeStruct((14, HEP), jnp.int32),
              jax.ShapeDtypeStruct((2, VR), jnp.float32)],
    mesh=_MESH,
    compiler_params=_SC_PARAMS,
    scratch_types=[
        pltpu.VMEM((VR,), jnp.float32),
        pltpu.VMEM((K * EW,), jnp.int32),
        pltpu.VMEM((K, NWIN, WIN), jnp.int32),
        pltpu.VMEM((14 * EW,), jnp.int32),
        pltpu.VMEM((K * EW,), jnp.float32),
        pltpu.VMEM_SHARED((VR,), jnp.float32),
    ],
)
def _sc_edges(hed2_hbm, q_hbm, z1_hbm, src_out, deg_out,
              q_v, vf, vb, of, dgf, deg_sh):
    c = lax.axis_index("c")
    s = lax.axis_index("s")
    wid = s * 2 + c
    base = wid * EW

    pltpu.sync_copy(z1_hbm.at[pl.ds(s * 640, 640)],
                    deg_sh.at[pl.ds(s * 640, 640)])
    plsc.subcore_barrier()

    for j in range(K):
        pltpu.sync_copy(hed2_hbm.at[j, pl.ds(base, EW)],
                        vf.at[pl.ds(j * EW, EW)])
        for w in range(NWIN):
            pltpu.sync_copy(hed2_hbm.at[j, pl.ds(base + w * WIN, WIN)],
                            vb.at[j, w])
    pltpu.sync_copy(q_hbm, q_v)

    lane = lax.iota(jnp.int32, 16)

    @pl.loop(0, EW // 16)
    def _grp(g):
        off = g * 16
        sl = pl.ds(off, 16)
        v = [vf[pl.ds(j * EW + off, 16)] for j in range(K)]
        p = [plsc.load_gather(q_v, [v[j]]) for j in range(K)]
        pmax = jnp.maximum(jnp.maximum(p[0], p[1]), jnp.maximum(p[2], p[3]))
        pmin = jnp.minimum(jnp.minimum(p[0], p[1]), jnp.minimum(p[2], p[3]))
        isx = [p[j] == pmax for j in range(K)]
        isn = [p[j] == pmin for j in range(K)]
        imax = [isx[0],
                isx[1] & ~isx[0],
                isx[2] & ~(isx[0] | isx[1]),
                isx[3] & ~(isx[0] | isx[1] | isx[2])]
        imin = [isn[0],
                isn[1] & ~isn[0],
                isn[2] & ~(isn[0] | isn[1]),
                isn[3] & ~(isn[0] | isn[1] | isn[2])]
        se = jnp.where(imax[0], v[0],
                       jnp.where(imax[1], v[1],
                                 jnp.where(imax[2], v[2], v[3])))
        ie = jnp.where(imin[0], v[0],
                       jnp.where(imin[1], v[1],
                                 jnp.where(imin[2], v[2], v[3])))
        med = [(v[j] != se) & (v[j] != ie) for j in range(K)]
        medf = [med[j].astype(jnp.float32) for j in range(K)]
        nmed = medf[0] + medf[1] + medf[2] + medf[3]
        dummy = V + ((off + wid * 16 + lane) & DUMMY_MASK)
        of[sl] = se
        of[pl.ds(EW + off, 16)] = ie
        for j in range(K):
            of[pl.ds((2 + j) * EW + off, 16)] = jnp.where(med[j], se, dummy)
            of[pl.ds((6 + j) * EW + off, 16)] = jnp.where(med[j], ie, dummy)
            of[pl.ds((10 + j) * EW + off, 16)] = jnp.where(med[j], v[j], dummy)
            extf = imax[j].astype(jnp.float32) + imin[j].astype(jnp.float32)
            dgf[pl.ds(j * EW + off, 16)] = W0 * (2.0 * medf[j]
                                                 + extf * (1.0 + nmed))

    for r in range(14):
        pltpu.sync_copy(of.at[pl.ds(r * EW, EW)],
                        src_out.at[r, pl.ds(base, EW)])
    for j in range(K):
        for w in range(NWIN):
            pltpu.sync_copy(dgf.at[pl.ds(j * EW + w * WIN, WIN)],
                            deg_sh.at[vb.at[j, w]], add=True)

    plsc.subcore_barrier()
    pltpu.sync_copy(deg_sh.at[pl.ds(s * 640, 640)],
                    deg_out.at[c, pl.ds(s * 640, 640)])


_PAIRS = [(0, (1,)), (1, (0,))]
for _j in range(K):
    _PAIRS += [(2 + _j, (14 + _j,)), (6 + _j, (14 + _j,)), (10 + _j, (0, 1))]

_NBUF = 4


def _make_scatter(d):

    @functools.partial(
        pl.kernel,
        out_type=jax.ShapeDtypeStruct((2, VR, d), jnp.float32),
        mesh=_MESH,
        compiler_params=_SC_PARAMS,
        scratch_types=(
            [pltpu.VMEM((18, NWIN, WIN), jnp.int32)]
            + [pltpu.VMEM((WIN, d), jnp.float32)] * _NBUF
            + [pltpu.VMEM_SHARED((VR, d), jnp.float32)]
            + [pltpu.SemaphoreType.DMA] * (2 * _NBUF)
        ),
    )
    def _scatter(g_hbm, src_hbm, hed_hbm, z_hbm, t_out, idx,
                 r0, r1, r2, r3, t_sh, *sems):
        c = lax.axis_index("c")
        s = lax.axis_index("s")
        wid = s * 2 + c
        bufs = (r0, r1, r2, r3)
        gsem = sems[:_NBUF]
        ssem = sems[_NBUF:]

        pltpu.sync_copy(z_hbm.at[pl.ds(s * 640, 640), :],
                        t_sh.at[pl.ds(s * 640, 640), :])
        plsc.subcore_barrier()

        pltpu.sync_copy(src_hbm.at[:, wid], idx.at[pl.ds(0, 14)])
        pltpu.sync_copy(hed_hbm.at[:, wid], idx.at[pl.ds(14, K)])

        for (sr, drs) in _PAIRS:
            for b in range(_NBUF):
                pltpu.async_copy(g_hbm.at[idx.at[sr, b]], bufs[b], gsem[b])

            @pl.loop(0, NWIN // _NBUF)
            def _win(q):
                for b in range(_NBUF):
                    w = q * _NBUF + b
                    pltpu.make_async_copy(g_hbm.at[idx.at[sr, w]], bufs[b],
                                          gsem[b]).wait()
                    for dr in drs:
                        pltpu.async_copy(bufs[b], t_sh.at[idx.at[dr, w]],
                                         ssem[b], add=True)
                    for dr in drs:
                        pltpu.make_async_copy(bufs[b],
                                              t_sh.at[idx.at[dr, w]],
                                              ssem[b]).wait()

                    @pl.when(w + _NBUF < NWIN)
                    def _():
                        pltpu.async_copy(g_hbm.at[idx.at[sr, w + _NBUF]],
                                         bufs[b], gsem[b])

        plsc.subcore_barrier()
        pltpu.sync_copy(t_sh.at[pl.ds(s * 640, 640), :],
                        t_out.at[c, pl.ds(s * 640, 640), :])

    return _scatter


_scatter_d1 = _make_scatter(D1)
_scatter_d2 = _make_scatter(D2)


def kernel(H, hyperedges, rv, W1, b1, W2, b2):
    f32 = jnp.float32
    h_pad = jnp.zeros((VR, DIN), f32).at[0:V, :].set(H)
    padv = (V + (jnp.arange(NPAD, dtype=jnp.int32) % (VR - V)))[:, None]
    he_pad = jnp.concatenate(
        [hyperedges.astype(jnp.int32), jnp.broadcast_to(padv, (NPAD, K))], 0)
    hed2 = he_pad.T
    hed4 = hed2.reshape(K, NW, NWIN, WIN)
    w2p = jnp.zeros((D1, D2), f32).at[:, 0:NCLS].set(W2)
    b2p = jnp.zeros((D2,), f32).at[0:NCLS].set(b2)
    z1 = jnp.zeros((VR,), f32)
    zd1 = jnp.zeros((VR, D1), f32)
    zd2 = jnp.zeros((VR, D2), f32)

    q, hw1 = _tc1(h_pad, W1, rv)
    src_all, deg_part = _sc_edges(hed2, q, z1)
    src4 = src_all.reshape(14, NW, NWIN, WIN)
    ds, dinv, g1 = _tc2(deg_part, hw1)
    t1 = _scatter_d1(g1, src4, hed4, zd1)
    hw2, g2 = _tc3(t1, hw1, ds, dinv, b1, w2p)
    t2 = _scatter_d2(g2, src4, hed4, zd2)
    return _tc4(t2, hw2, ds, dinv, b2p)


# speedup vs baseline: 119.7896x; 1.3275x over previous
import functools

import jax
import jax.numpy as jnp
from jax import lax
from jax.experimental import pallas as pl
from jax.experimental.pallas import tpu as pltpu
from jax.experimental.pallas import tpu_sc as plsc

V = 10000
DIN = 128
D1 = 16
NCLS = 40
D2 = 48
K = 4
W0 = 1.0 / 5.0

HE = 80000
NW = 32
WIN = 128
NWIN = 20
EW = WIN * NWIN
HEP = NW * EW
NPAD = HEP - HE

VR = 10240
VE = 12288
DUMMY_MASK = 2047

_HIGH = lax.Precision.HIGHEST


def _tc1_body(h_ref, w1_ref, rv_ref, q_ref, hw1_ref):
    H = h_ref[...]
    rv = rv_ref[...]
    q_ref[...] = jnp.sum(H * rv[None, :], axis=1)
    hw1_ref[...] = lax.dot_general(H, w1_ref[...], (((1,), (0,)), ((), ())),
                                   precision=_HIGH)


def _tc1(h_pad, w1, rv):
    return pl.pallas_call(
        _tc1_body,
        out_shape=[jax.ShapeDtypeStruct((VR,), jnp.float32),
                   jax.ShapeDtypeStruct((VR, D1), jnp.float32)],
    )(h_pad, w1, rv)


def _tc2_body(degp_ref, hw1_ref, ds_ref, dinv_ref, g1_ref):
    deg = degp_ref[0, :] + degp_ref[1, :] + 1.0
    ds = lax.rsqrt(deg)
    dinv = 1.0 / deg
    ds_ref[...] = ds
    dinv_ref[...] = dinv
    g1_ref[0:VR, :] = W0 * ds[:, None] * hw1_ref[...]
    g1_ref[VR:VE, :] = jnp.zeros((VE - VR, D1), jnp.float32)


def _tc2(deg_part, hw1):
    return pl.pallas_call(
        _tc2_body,
        out_shape=[jax.ShapeDtypeStruct((VR,), jnp.float32),
                   jax.ShapeDtypeStruct((VR,), jnp.float32),
                   jax.ShapeDtypeStruct((VE, D1), jnp.float32)],
    )(deg_part, hw1)


def _tc3_body(t1_ref, hw1_ref, ds_ref, dinv_ref, b1_ref, w2_ref,
              hw2_ref, g2_ref):
    T = t1_ref[0] + t1_ref[1]
    ds = ds_ref[...]
    dinv = dinv_ref[...]
    pre = ds[:, None] * T + dinv[:, None] * hw1_ref[...] + b1_ref[...][None, :]
    h = jnp.maximum(pre, 0.0)
    rowid = lax.broadcasted_iota(jnp.int32, (VR, D1), 0)
    h = jnp.where(rowid < V, h, 0.0)
    hw2 = lax.dot_general(h, w2_ref[...], (((1,), (0,)), ((), ())),
                          precision=_HIGH)
    hw2_ref[...] = hw2
    g2_ref[0:VR, :] = W0 * ds[:, None] * hw2
    g2_ref[VR:VE, :] = jnp.zeros((VE - VR, D2), jnp.float32)


def _tc3(t1_part, hw1, ds, dinv, b1, w2p):
    return pl.pallas_call(
        _tc3_body,
        out_shape=[jax.ShapeDtypeStruct((VR, D2), jnp.float32),
                   jax.ShapeDtypeStruct((VE, D2), jnp.float32)],
    )(t1_part, hw1, ds, dinv, b1, w2p)


def _tc4_body(t2_ref, hw2_ref, ds_ref, dinv_ref, b2_ref, out_ref):
    T = t2_ref[0] + t2_ref[1]
    pre = (ds_ref[...][:, None] * T + dinv_ref[...][:, None] * hw2_ref[...]
           + b2_ref[...][None, :])
    o = jnp.maximum(pre, 0.0)
    logits = o[0:V, 0:NCLS]
    m = jnp.max(logits, axis=1, keepdims=True)
    e = jnp.exp(logits - m)
    lse = jnp.log(jnp.sum(e, axis=1, keepdims=True)) + m
    out_ref[...] = logits - lse


def _tc4(t2_part, hw2, ds, dinv, b2p):
    return pl.pallas_call(
        _tc4_body,
        out_shape=jax.ShapeDtypeStruct((V, NCLS), jnp.float32),
    )(t2_part, hw2, ds, dinv, b2p)


_MESH = plsc.VectorSubcoreMesh(core_axis_name="c", subcore_axis_name="s")
_SC_PARAMS = pltpu.CompilerParams(needs_layout_passes=False,
                                  use_tc_tiling_on_sc=False)


@functools.partial(
    pl.kernel,
    out_type=[jax.ShapeDtypeStruct((10, HEP), jnp.int32),
              jax.ShapeDtypeStruct((2, VR), jnp.float32)],
    mesh=_MESH,
    compiler_params=_SC_PARAMS,
    scratch_types=[
        pltpu.VMEM((VR,), jnp.float32),
        pltpu.VMEM((K * EW,), jnp.int32),
        pltpu.VMEM((K, NWIN, WIN), jnp.int32),
        pltpu.VMEM((10 * EW,), jnp.int32),
        pltpu.VMEM((K * EW,), jnp.float32),
        pltpu.VMEM_SHARED((VR,), jnp.float32),
        pltpu.SemaphoreType.DMA,
    ],
)
def _sc_edges(hed2_hbm, q_hbm, z1_hbm, src_out, deg_out,
              q_v, vf, vb, of, dgf, deg_sh, sem):
    c = lax.axis_index("c")
    s = lax.axis_index("s")
    wid = s * 2 + c
    base = wid * EW

    pltpu.sync_copy(z1_hbm.at[pl.ds(s * 640, 640)],
                    deg_sh.at[pl.ds(s * 640, 640)])
    plsc.subcore_barrier()

    for j in range(K):
        pltpu.sync_copy(hed2_hbm.at[j, pl.ds(base, EW)],
                        vf.at[pl.ds(j * EW, EW)])
        for w in range(NWIN):
            pltpu.sync_copy(hed2_hbm.at[j, pl.ds(base + w * WIN, WIN)],
                            vb.at[j, w])
    pltpu.sync_copy(q_hbm, q_v)

    lane = lax.iota(jnp.int32, 16)

    @pl.loop(0, EW // 16)
    def _grp(g):
        off = g * 16
        sl = pl.ds(off, 16)
        v = [vf[pl.ds(j * EW + off, 16)] for j in range(K)]
        p = [plsc.load_gather(q_v, [v[j]]) for j in range(K)]
        pmax = jnp.maximum(jnp.maximum(p[0], p[1]), jnp.maximum(p[2], p[3]))
        pmin = jnp.minimum(jnp.minimum(p[0], p[1]), jnp.minimum(p[2], p[3]))
        isx = [p[j] == pmax for j in range(K)]
        isn = [p[j] == pmin for j in range(K)]
        imax = [isx[0],
                isx[1] & ~isx[0],
                isx[2] & ~(isx[0] | isx[1]),
                isx[3] & ~(isx[0] | isx[1] | isx[2])]
        imin = [isn[0],
                isn[1] & ~isn[0],
                isn[2] & ~(isn[0] | isn[1]),
                isn[3] & ~(isn[0] | isn[1] | isn[2])]
        se = jnp.where(imax[0], v[0],
                       jnp.where(imax[1], v[1],
                                 jnp.where(imax[2], v[2], v[3])))
        ie = jnp.where(imin[0], v[0],
                       jnp.where(imin[1], v[1],
                                 jnp.where(imin[2], v[2], v[3])))
        med = [(v[j] != se) & (v[j] != ie) for j in range(K)]
        medf = [med[j].astype(jnp.float32) for j in range(K)]
        nmed = medf[0] + medf[1] + medf[2] + medf[3]
        f1 = [med[0],
              med[1] & ~med[0],
              med[2] & ~(med[0] | med[1]),
              med[3] & ~(med[0] | med[1] | med[2])]
        f2_1 = med[1] & med[0]
        f2_2 = med[2] & (med[0] ^ med[1])
        f2_3 = med[3] & (med[0] ^ med[1] ^ med[2])
        has1 = med[0] | med[1] | med[2] | med[3]
        has2 = f2_1 | f2_2 | f2_3
        m1 = jnp.where(f1[0], v[0],
                       jnp.where(f1[1], v[1],
                                 jnp.where(f1[2], v[2], v[3])))
        m2 = jnp.where(f2_1, v[1], jnp.where(f2_2, v[2], v[3]))
        dummy = V + ((off + wid * 16 + lane) & DUMMY_MASK)
        dummy2 = V + ((off + wid * 16 + lane + 1024) & DUMMY_MASK)
        of[sl] = se
        of[pl.ds(EW + off, 16)] = ie
        of[pl.ds(2 * EW + off, 16)] = jnp.where(has1, se, dummy)
        of[pl.ds(3 * EW + off, 16)] = jnp.where(has1, ie, dummy2)
        of[pl.ds(4 * EW + off, 16)] = jnp.where(has1, m1, dummy)
        of[pl.ds(5 * EW + off, 16)] = jnp.where(has1, m1, se)
        of[pl.ds(6 * EW + off, 16)] = jnp.where(has2, se, dummy2)
        of[pl.ds(7 * EW + off, 16)] = jnp.where(has2, ie, dummy)
        of[pl.ds(8 * EW + off, 16)] = jnp.where(has2, m2, dummy2)
        of[pl.ds(9 * EW + off, 16)] = jnp.where(has2, m2, se)
        for j in range(K):
            extf = imax[j].astype(jnp.float32) + imin[j].astype(jnp.float32)
            dgf[pl.ds(j * EW + off, 16)] = W0 * (2.0 * medf[j]
                                                 + extf * (1.0 + nmed))

    for r in range(10):
        pltpu.sync_copy(of.at[pl.ds(r * EW, EW)],
                        src_out.at[r, pl.ds(base, EW)])
    for j in range(K):
        for w in range(NWIN):
            pltpu.sync_copy(dgf.at[pl.ds(j * EW + w * WIN, WIN)],
                            deg_sh.at[vb.at[j, w]], add=True)

    plsc.subcore_barrier()
    pltpu.sync_copy(deg_sh.at[pl.ds(s * 640, 640)],
                    deg_out.at[c, pl.ds(s * 640, 640)])


_PAIRS = [(0, (1,)), (1, (0,)),
          (2, (5,)), (3, (5,)), (4, (0, 1)),
          (6, (9,)), (7, (9,)), (8, (0, 1))]

_NBUF = 4


def _make_scatter(d):

    @functools.partial(
        pl.kernel,
        out_type=jax.ShapeDtypeStruct((2, VR, d), jnp.float32),
        mesh=_MESH,
        compiler_params=_SC_PARAMS,
        scratch_types=(
            [pltpu.VMEM((10, NWIN, WIN), jnp.int32)]
            + [pltpu.VMEM((WIN, d), jnp.float32)] * _NBUF
            + [pltpu.VMEM_SHARED((VR, d), jnp.float32)]
            + [pltpu.SemaphoreType.DMA] * (2 * _NBUF)
        ),
    )
    def _scatter(g_hbm, src_hbm, z_hbm, t_out, idx,
                 r0, r1, r2, r3, t_sh, *sems):
        c = lax.axis_index("c")
        s = lax.axis_index("s")
        wid = s * 2 + c
        bufs = (r0, r1, r2, r3)
        gsem = sems[:_NBUF]
        ssem = sems[_NBUF:]

        pltpu.sync_copy(z_hbm.at[pl.ds(s * 640, 640), :],
                        t_sh.at[pl.ds(s * 640, 640), :])
        plsc.subcore_barrier()

        pltpu.sync_copy(src_hbm.at[:, wid], idx)

        for (sr, drs) in _PAIRS:
            for b in range(_NBUF):
                pltpu.async_copy(g_hbm.at[idx.at[sr, b]], bufs[b], gsem[b])

            @pl.loop(0, NWIN // _NBUF)
            def _win(q):
                for b in range(_NBUF):
                    w = q * _NBUF + b
                    pltpu.make_async_copy(g_hbm.at[idx.at[sr, w]], bufs[b],
                                          gsem[b]).wait()
                    for dr in drs:
                        pltpu.async_copy(bufs[b], t_sh.at[idx.at[dr, w]],
                                         ssem[b], add=True)
                    for dr in drs:
                        pltpu.make_async_copy(bufs[b],
                                              t_sh.at[idx.at[dr, w]],
                                              ssem[b]).wait()

                    @pl.when(w + _NBUF < NWIN)
                    def _():
                        pltpu.async_copy(g_hbm.at[idx.at[sr, w + _NBUF]],
                                         bufs[b], gsem[b])

        plsc.subcore_barrier()
        pltpu.sync_copy(t_sh.at[pl.ds(s * 640, 640), :],
                        t_out.at[c, pl.ds(s * 640, 640), :])

    return _scatter


_scatter_d1 = _make_scatter(D1)
_scatter_d2 = _make_scatter(D2)


def kernel(H, hyperedges, rv, W1, b1, W2, b2):
    f32 = jnp.float32
    h_pad = jnp.zeros((VR, DIN), f32).at[0:V, :].set(H)
    padv = (V + (jnp.arange(NPAD, dtype=jnp.int32) % (VR - V)))[:, None]
    he_pad = jnp.concatenate(
        [hyperedges.astype(jnp.int32), jnp.broadcast_to(padv, (NPAD, K))], 0)
    hed2 = he_pad.T
    w2p = jnp.zeros((D1, D2), f32).at[:, 0:NCLS].set(W2)
    b2p = jnp.zeros((D2,), f32).at[0:NCLS].set(b2)
    z1 = jnp.zeros((VR,), f32)
    zd1 = jnp.zeros((VR, D1), f32)
    zd2 = jnp.zeros((VR, D2), f32)

    q, hw1 = _tc1(h_pad, W1, rv)
    src_all, deg_part = _sc_edges(hed2, q, z1)
    src4 = src_all.reshape(10, NW, NWIN, WIN)
    ds, dinv, g1 = _tc2(deg_part, hw1)
    t1 = _scatter_d1(g1, src4, zd1)
    hw2, g2 = _tc3(t1, hw1, ds, dinv, b1, w2p)
    t2 = _scatter_d2(g2, src4, zd2)
    return _tc4(t2, hw2, ds, dinv, b2p)


# speedup vs baseline: 142.7674x; 1.1918x over previous
import functools

import jax
import jax.numpy as jnp
from jax import lax
from jax.experimental import pallas as pl
from jax.experimental.pallas import tpu as pltpu
from jax.experimental.pallas import tpu_sc as plsc

V = 10000
DIN = 128
D1 = 16
NCLS = 40
D2 = 40
K = 4
W0 = 1.0 / 5.0

HE = 80000
NW = 32
WIN = 128
NWIN = 20
EW = WIN * NWIN
HEP = NW * EW
NPAD = HEP - HE

VR = 10240
VE = 12288
DUMMY_MASK = 2047

_HIGH = lax.Precision.HIGHEST


def _tc1_body(h_ref, w1_ref, rv_ref, q_ref, hw1_ref):
    H = h_ref[...]
    rv = rv_ref[...]
    q_ref[...] = jnp.sum(H * rv[None, :], axis=1)
    hw1_ref[...] = lax.dot_general(H, w1_ref[...], (((1,), (0,)), ((), ())),
                                   precision=_HIGH)


def _tc1(h_pad, w1, rv):
    return pl.pallas_call(
        _tc1_body,
        out_shape=[jax.ShapeDtypeStruct((VR,), jnp.float32),
                   jax.ShapeDtypeStruct((VR, D1), jnp.float32)],
    )(h_pad, w1, rv)


def _tc2_body(degp_ref, hw1_ref, ds_ref, dinv_ref, g1_ref):
    deg = jnp.sum(degp_ref[...], axis=0) + 1.0
    ds = lax.rsqrt(deg)
    dinv = 1.0 / deg
    ds_ref[...] = ds
    dinv_ref[...] = dinv
    g1_ref[0:VR, :] = W0 * ds[:, None] * hw1_ref[...]
    g1_ref[VR:VE, :] = jnp.zeros((VE - VR, D1), jnp.float32)


def _tc2(deg_part, hw1):
    return pl.pallas_call(
        _tc2_body,
        out_shape=[jax.ShapeDtypeStruct((VR,), jnp.float32),
                   jax.ShapeDtypeStruct((VR,), jnp.float32),
                   jax.ShapeDtypeStruct((VE, D1), jnp.float32)],
    )(deg_part, hw1)


def _tc3_body(t1_ref, hw1_ref, ds_ref, dinv_ref, b1_ref, w2_ref,
              hw2_ref, g2_ref):
    T = t1_ref[0] + t1_ref[1]
    ds = ds_ref[...]
    dinv = dinv_ref[...]
    pre = ds[:, None] * T + dinv[:, None] * hw1_ref[...] + b1_ref[...][None, :]
    h = jnp.maximum(pre, 0.0)
    rowid = lax.broadcasted_iota(jnp.int32, (VR, D1), 0)
    h = jnp.where(rowid < V, h, 0.0)
    hw2 = lax.dot_general(h, w2_ref[...], (((1,), (0,)), ((), ())),
                          precision=_HIGH)
    hw2_ref[...] = hw2
    g2_ref[0:VR, :] = W0 * ds[:, None] * hw2
    g2_ref[VR:VE, :] = jnp.zeros((VE - VR, D2), jnp.float32)


def _tc3(t1_part, hw1, ds, dinv, b1, w2p):
    return pl.pallas_call(
        _tc3_body,
        out_shape=[jax.ShapeDtypeStruct((VR, D2), jnp.float32),
                   jax.ShapeDtypeStruct((VE, D2), jnp.float32)],
    )(t1_part, hw1, ds, dinv, b1, w2p)


def _tc4_body(t2_ref, hw2_ref, ds_ref, dinv_ref, b2_ref, out_ref):
    T = t2_ref[0] + t2_ref[1]
    pre = (ds_ref[...][:, None] * T + dinv_ref[...][:, None] * hw2_ref[...]
           + b2_ref[...][None, :])
    o = jnp.maximum(pre, 0.0)
    logits = o[0:V, :]
    m = jnp.max(logits, axis=1, keepdims=True)
    e = jnp.exp(logits - m)
    lse = jnp.log(jnp.sum(e, axis=1, keepdims=True)) + m
    out_ref[...] = logits - lse


def _tc4(t2_part, hw2, ds, dinv, b2p):
    return pl.pallas_call(
        _tc4_body,
        out_shape=jax.ShapeDtypeStruct((V, NCLS), jnp.float32),
    )(t2_part, hw2, ds, dinv, b2p)


_MESH = plsc.VectorSubcoreMesh(core_axis_name="c", subcore_axis_name="s")
_SC_PARAMS = pltpu.CompilerParams(needs_layout_passes=False,
                                  use_tc_tiling_on_sc=False)


@functools.partial(
    pl.kernel,
    out_type=[jax.ShapeDtypeStruct((10, HEP), jnp.int32),
              jax.ShapeDtypeStruct((NW, VR), jnp.float32)],
    mesh=_MESH,
    compiler_params=_SC_PARAMS,
    scratch_types=[
        pltpu.VMEM((VR,), jnp.float32),
        pltpu.VMEM((K * EW,), jnp.int32),
        pltpu.VMEM((10 * EW,), jnp.int32),
        pltpu.VMEM((VR,), jnp.float32),
        pltpu.SemaphoreType.DMA,
    ],
)
def _sc_edges(hed2_hbm, q_hbm, z1_hbm, src_out, deg_out,
              q_v, vf, of, deg_l, sem):
    c = lax.axis_index("c")
    s = lax.axis_index("s")
    wid = s * 2 + c
    base = wid * EW

    pltpu.sync_copy(z1_hbm, deg_l)
    for j in range(K):
        pltpu.sync_copy(hed2_hbm.at[j, pl.ds(base, EW)],
                        vf.at[pl.ds(j * EW, EW)])
    pltpu.sync_copy(q_hbm, q_v)

    lane = lax.iota(jnp.int32, 16)

    @pl.loop(0, EW // 16)
    def _grp(g):
        off = g * 16
        sl = pl.ds(off, 16)
        v = [vf[pl.ds(j * EW + off, 16)] for j in range(K)]
        p = [plsc.load_gather(q_v, [v[j]]) for j in range(K)]
        pmax = jnp.maximum(jnp.maximum(p[0], p[1]), jnp.maximum(p[2], p[3]))
        pmin = jnp.minimum(jnp.minimum(p[0], p[1]), jnp.minimum(p[2], p[3]))
        isx = [p[j] == pmax for j in range(K)]
        isn = [p[j] == pmin for j in range(K)]
        imax = [isx[0],
                isx[1] & ~isx[0],
                isx[2] & ~(isx[0] | isx[1]),
                isx[3] & ~(isx[0] | isx[1] | isx[2])]
        imin = [isn[0],
                isn[1] & ~isn[0],
                isn[2] & ~(isn[0] | isn[1]),
                isn[3] & ~(isn[0] | isn[1] | isn[2])]
        se = jnp.where(imax[0], v[0],
                       jnp.where(imax[1], v[1],
                                 jnp.where(imax[2], v[2], v[3])))
        ie = jnp.where(imin[0], v[0],
                       jnp.where(imin[1], v[1],
                                 jnp.where(imin[2], v[2], v[3])))
        med = [(v[j] != se) & (v[j] != ie) for j in range(K)]
        medf = [med[j].astype(jnp.float32) for j in range(K)]
        nmed = medf[0] + medf[1] + medf[2] + medf[3]
        f1 = [med[0],
              med[1] & ~med[0],
              med[2] & ~(med[0] | med[1]),
              med[3] & ~(med[0] | med[1] | med[2])]
        f2_1 = med[1] & med[0]
        f2_2 = med[2] & (med[0] ^ med[1])
        f2_3 = med[3] & (med[0] ^ med[1] ^ med[2])
        has1 = med[0] | med[1] | med[2] | med[3]
        has2 = f2_1 | f2_2 | f2_3
        m1 = jnp.where(f1[0], v[0],
                       jnp.where(f1[1], v[1],
                                 jnp.where(f1[2], v[2], v[3])))
        m2 = jnp.where(f2_1, v[1], jnp.where(f2_2, v[2], v[3]))
        dummy = V + ((off + wid * 16 + lane) & DUMMY_MASK)
        dummy2 = V + ((off + wid * 16 + lane + 1024) & DUMMY_MASK)
        of[sl] = se
        of[pl.ds(EW + off, 16)] = ie
        of[pl.ds(2 * EW + off, 16)] = jnp.where(has1, se, dummy)
        of[pl.ds(3 * EW + off, 16)] = jnp.where(has1, ie, dummy2)
        of[pl.ds(4 * EW + off, 16)] = jnp.where(has1, m1, dummy)
        of[pl.ds(5 * EW + off, 16)] = jnp.where(has1, m1, se)
        of[pl.ds(6 * EW + off, 16)] = jnp.where(has2, se, dummy2)
        of[pl.ds(7 * EW + off, 16)] = jnp.where(has2, ie, dummy)
        of[pl.ds(8 * EW + off, 16)] = jnp.where(has2, m2, dummy2)
        of[pl.ds(9 * EW + off, 16)] = jnp.where(has2, m2, se)
        for j in range(K):
            extf = imax[j].astype(jnp.float32) + imin[j].astype(jnp.float32)
            degv = W0 * (2.0 * medf[j] + extf * (1.0 + nmed))
            plsc.addupdate_scatter(deg_l, [v[j]], degv)

    for r in range(10):
        pltpu.sync_copy(of.at[pl.ds(r * EW, EW)],
                        src_out.at[r, pl.ds(base, EW)])
    pltpu.sync_copy(deg_l, deg_out.at[wid])


_PAIRS = [(0, (1,)), (1, (0,)),
          (2, (5,)), (3, (5,)), (4, (0, 1)),
          (6, (9,)), (7, (9,)), (8, (0, 1))]

_NBUF = 4


def _make_scatter(d):

    @functools.partial(
        pl.kernel,
        out_type=jax.ShapeDtypeStruct((2, VR, d), jnp.float32),
        mesh=_MESH,
        compiler_params=_SC_PARAMS,
        scratch_types=(
            [pltpu.VMEM((10, NWIN, WIN), jnp.int32)]
            + [pltpu.VMEM((WIN, d), jnp.float32)] * _NBUF
            + [pltpu.VMEM_SHARED((VR, d), jnp.float32)]
            + [pltpu.SemaphoreType.DMA] * (2 * _NBUF)
        ),
    )
    def _scatter(g_hbm, src_hbm, z_hbm, t_out, idx,
                 r0, r1, r2, r3, t_sh, *sems):
        c = lax.axis_index("c")
        s = lax.axis_index("s")
        wid = s * 2 + c
        bufs = (r0, r1, r2, r3)
        gsem = sems[:_NBUF]
        ssem = sems[_NBUF:]

        pltpu.sync_copy(z_hbm.at[pl.ds(s * 640, 640), :],
                        t_sh.at[pl.ds(s * 640, 640), :])
        plsc.subcore_barrier()

        pltpu.sync_copy(src_hbm.at[:, wid], idx)

        for (sr, drs) in _PAIRS:
            for b in range(_NBUF):
                pltpu.async_copy(g_hbm.at[idx.at[sr, b]], bufs[b], gsem[b])

            @pl.loop(0, NWIN // _NBUF)
            def _win(q):
                for b in range(_NBUF):
                    w = q * _NBUF + b
                    pltpu.make_async_copy(g_hbm.at[idx.at[sr, w]], bufs[b],
                                          gsem[b]).wait()
                    for dr in drs:
                        pltpu.async_copy(bufs[b], t_sh.at[idx.at[dr, w]],
                                         ssem[b], add=True)
                    for dr in drs:
                        pltpu.make_async_copy(bufs[b],
                                              t_sh.at[idx.at[dr, w]],
                                              ssem[b]).wait()

                    @pl.when(w + _NBUF < NWIN)
                    def _():
                        pltpu.async_copy(g_hbm.at[idx.at[sr, w + _NBUF]],
                                         bufs[b], gsem[b])

        plsc.subcore_barrier()
        pltpu.sync_copy(t_sh.at[pl.ds(s * 640, 640), :],
                        t_out.at[c, pl.ds(s * 640, 640), :])

    return _scatter


_scatter_d1 = _make_scatter(D1)
_scatter_d2 = _make_scatter(D2)


def kernel(H, hyperedges, rv, W1, b1, W2, b2):
    f32 = jnp.float32
    h_pad = jnp.zeros((VR, DIN), f32).at[0:V, :].set(H)
    padv = (V + (jnp.arange(NPAD, dtype=jnp.int32) % (VR - V)))[:, None]
    he_pad = jnp.concatenate(
        [hyperedges.astype(jnp.int32), jnp.broadcast_to(padv, (NPAD, K))], 0)
    hed2 = he_pad.T
    w2p = W2.astype(f32)
    b2p = b2.astype(f32)
    z1 = jnp.zeros((VR,), f32)
    zd1 = jnp.zeros((VR, D1), f32)
    zd2 = jnp.zeros((VR, D2), f32)

    q, hw1 = _tc1(h_pad, W1, rv)
    src_all, deg_part = _sc_edges(hed2, q, z1)
    src4 = src_all.reshape(10, NW, NWIN, WIN)
    ds, dinv, g1 = _tc2(deg_part, hw1)
    t1 = _scatter_d1(g1, src4, zd1)
    hw2, g2 = _tc3(t1, hw1, ds, dinv, b1, w2p)
    t2 = _scatter_d2(g2, src4, zd2)
    return _tc4(t2, hw2, ds, dinv, b2p)


# speedup vs baseline: 171.5263x; 1.2014x over previous
import functools

import jax
import jax.numpy as jnp
from jax import lax
from jax.experimental import pallas as pl
from jax.experimental.pallas import tpu as pltpu
from jax.experimental.pallas import tpu_sc as plsc

V = 10000
DIN = 128
D1 = 16
NCLS = 40
D2 = 48
K = 4
W0 = 1.0 / 5.0

HE = 80000
NW = 32
WIN = 128
NWIN = 20
EW = WIN * NWIN
HEP = NW * EW
NPAD = HEP - HE

VR = 10240
VE = 12288
DUMMY_MASK = 2047

_HIGH = lax.Precision.HIGHEST


def _tc1_body(h_ref, w1_ref, rv_ref, q_ref, hw1_ref):
    H = h_ref[...]
    rv = rv_ref[...]
    q_ref[...] = jnp.sum(H * rv[None, :], axis=1)
    hw1_ref[...] = lax.dot_general(H, w1_ref[...], (((1,), (0,)), ((), ())),
                                   precision=_HIGH)


def _tc1(h_pad, w1, rv):
    return pl.pallas_call(
        _tc1_body,
        out_shape=[jax.ShapeDtypeStruct((VR,), jnp.float32),
                   jax.ShapeDtypeStruct((VR, D1), jnp.float32)],
    )(h_pad, w1, rv)


def _tc2_body(degp_ref, hw1_ref, ds_ref, dinv_ref, g1_ref):
    deg = jnp.sum(degp_ref[...], axis=0) + 1.0
    ds = lax.rsqrt(deg)
    dinv = 1.0 / deg
    ds_ref[...] = ds
    dinv_ref[...] = dinv
    g1_ref[0:VR, :] = W0 * ds[:, None] * hw1_ref[...]
    g1_ref[VR:VE, :] = jnp.zeros((VE - VR, D1), jnp.float32)


def _tc2(deg_part, hw1):
    return pl.pallas_call(
        _tc2_body,
        out_shape=[jax.ShapeDtypeStruct((VR,), jnp.float32),
                   jax.ShapeDtypeStruct((VR,), jnp.float32),
                   jax.ShapeDtypeStruct((VE, D1), jnp.float32)],
    )(deg_part, hw1)


def _tc3_body(t1_ref, hw1_ref, ds_ref, dinv_ref, b1_ref, w2_ref,
              hw2_ref, g2_ref):
    T = t1_ref[0] + t1_ref[1]
    ds = ds_ref[...]
    dinv = dinv_ref[...]
    pre = ds[:, None] * T + dinv[:, None] * hw1_ref[...] + b1_ref[...][None, :]
    h = jnp.maximum(pre, 0.0)
    rowid = lax.broadcasted_iota(jnp.int32, (VR, D1), 0)
    h = jnp.where(rowid < V, h, 0.0)
    hw2 = lax.dot_general(h, w2_ref[...], (((1,), (0,)), ((), ())),
                          precision=_HIGH)
    hw2_ref[...] = hw2
    g2_ref[0:VR, :] = W0 * ds[:, None] * hw2
    g2_ref[VR:VE, :] = jnp.zeros((VE - VR, D2), jnp.float32)


def _tc3(t1_part, hw1, ds, dinv, b1, w2p):
    return pl.pallas_call(
        _tc3_body,
        out_shape=[jax.ShapeDtypeStruct((VR, D2), jnp.float32),
                   jax.ShapeDtypeStruct((VE, D2), jnp.float32)],
    )(t1_part, hw1, ds, dinv, b1, w2p)


def _tc4_body(t2_ref, hw2_ref, ds_ref, dinv_ref, b2_ref, out_ref):
    T = t2_ref[0] + t2_ref[1]
    pre = (ds_ref[...][:, None] * T + dinv_ref[...][:, None] * hw2_ref[...]
           + b2_ref[...][None, :])
    o = jnp.maximum(pre, 0.0)
    logits = o[0:V, 0:NCLS]
    m = jnp.max(logits, axis=1, keepdims=True)
    e = jnp.exp(logits - m)
    lse = jnp.log(jnp.sum(e, axis=1, keepdims=True)) + m
    out_ref[...] = logits - lse


def _tc4(t2_part, hw2, ds, dinv, b2p):
    return pl.pallas_call(
        _tc4_body,
        out_shape=jax.ShapeDtypeStruct((V, NCLS), jnp.float32),
    )(t2_part, hw2, ds, dinv, b2p)


_MESH = plsc.VectorSubcoreMesh(core_axis_name="c", subcore_axis_name="s")
_SC_PARAMS = pltpu.CompilerParams(needs_layout_passes=False,
                                  use_tc_tiling_on_sc=False)


@functools.partial(
    pl.kernel,
    out_type=[jax.ShapeDtypeStruct((6, HEP), jnp.int32),
              jax.ShapeDtypeStruct((NW, VR), jnp.float32)],
    mesh=_MESH,
    compiler_params=_SC_PARAMS,
    scratch_types=[
        pltpu.VMEM((VR,), jnp.float32),
        pltpu.VMEM((K * EW,), jnp.int32),
        pltpu.VMEM((6 * EW,), jnp.int32),
        pltpu.VMEM((VR,), jnp.float32),
        pltpu.SemaphoreType.DMA,
    ],
)
def _sc_edges(hed2_hbm, q_hbm, z1_hbm, src_out, deg_out,
              q_v, vf, of, deg_l, sem):
    c = lax.axis_index("c")
    s = lax.axis_index("s")
    wid = s * 2 + c
    base = wid * EW

    pltpu.sync_copy(z1_hbm, deg_l)
    for j in range(K):
        pltpu.sync_copy(hed2_hbm.at[j, pl.ds(base, EW)],
                        vf.at[pl.ds(j * EW, EW)])
    pltpu.sync_copy(q_hbm, q_v)

    lane = lax.iota(jnp.int32, 16)

    @pl.loop(0, EW // 16)
    def _grp(g):
        off = g * 16
        sl = pl.ds(off, 16)
        v = [vf[pl.ds(j * EW + off, 16)] for j in range(K)]
        p = [plsc.load_gather(q_v, [v[j]]) for j in range(K)]
        pmax = jnp.maximum(jnp.maximum(p[0], p[1]), jnp.maximum(p[2], p[3]))
        pmin = jnp.minimum(jnp.minimum(p[0], p[1]), jnp.minimum(p[2], p[3]))
        isx = [p[j] == pmax for j in range(K)]
        isn = [p[j] == pmin for j in range(K)]
        imax = [isx[0],
                isx[1] & ~isx[0],
                isx[2] & ~(isx[0] | isx[1]),
                isx[3] & ~(isx[0] | isx[1] | isx[2])]
        imin = [isn[0],
                isn[1] & ~isn[0],
                isn[2] & ~(isn[0] | isn[1]),
                isn[3] & ~(isn[0] | isn[1] | isn[2])]
        se = jnp.where(imax[0], v[0],
                       jnp.where(imax[1], v[1],
                                 jnp.where(imax[2], v[2], v[3])))
        ie = jnp.where(imin[0], v[0],
                       jnp.where(imin[1], v[1],
                                 jnp.where(imin[2], v[2], v[3])))
        med = [(v[j] != se) & (v[j] != ie) for j in range(K)]
        medf = [med[j].astype(jnp.float32) for j in range(K)]
        nmed = medf[0] + medf[1] + medf[2] + medf[3]
        f1 = [med[0],
              med[1] & ~med[0],
              med[2] & ~(med[0] | med[1]),
              med[3] & ~(med[0] | med[1] | med[2])]
        f2_1 = med[1] & med[0]
        f2_2 = med[2] & (med[0] ^ med[1])
        f2_3 = med[3] & (med[0] ^ med[1] ^ med[2])
        has1 = med[0] | med[1] | med[2] | med[3]
        has2 = f2_1 | f2_2 | f2_3
        m1 = jnp.where(f1[0], v[0],
                       jnp.where(f1[1], v[1],
                                 jnp.where(f1[2], v[2], v[3])))
        m2 = jnp.where(f2_1, v[1], jnp.where(f2_2, v[2], v[3]))
        dummy = V + ((off + wid * 16 + lane) & DUMMY_MASK)
        dummy2 = V + ((off + wid * 16 + lane + 1024) & DUMMY_MASK)
        trash = V + ((off + wid * 16 + lane) & 127)
        trash2 = V + ((off + wid * 16 + lane + 64) & 127)
        of[sl] = se
        of[pl.ds(EW + off, 16)] = ie
        of[pl.ds(2 * EW + off, 16)] = jnp.where(has1, m1, dummy)
        of[pl.ds(3 * EW + off, 16)] = jnp.where(has1, m1, trash)
        of[pl.ds(4 * EW + off, 16)] = jnp.where(has2, m2, dummy2)
        of[pl.ds(5 * EW + off, 16)] = jnp.where(has2, m2, trash2)
        for j in range(K):
            extf = imax[j].astype(jnp.float32) + imin[j].astype(jnp.float32)
            degv = W0 * (2.0 * medf[j] + extf * (1.0 + nmed))
            plsc.addupdate_scatter(deg_l, [v[j]], degv)

    for r in range(6):
        pltpu.sync_copy(of.at[pl.ds(r * EW, EW)],
                        src_out.at[r, pl.ds(base, EW)])
    pltpu.sync_copy(deg_l, deg_out.at[wid])


_NSLOT = 2


def _make_scatter(d):
    nch = d // 16

    @functools.partial(
        pl.kernel,
        out_type=jax.ShapeDtypeStruct((2, VR, d), jnp.float32),
        mesh=_MESH,
        compiler_params=_SC_PARAMS,
        scratch_types=(
            [pltpu.VMEM((6, NWIN, WIN), jnp.int32)]
            + [pltpu.VMEM((WIN, d), jnp.float32)] * (4 * _NSLOT)
            + [pltpu.VMEM_SHARED((VR, d), jnp.float32)]
            + [pltpu.SemaphoreType.DMA] * (2 * _NSLOT)
        ),
    )
    def _scatter(g_hbm, src_hbm, z_hbm, t_out, idx,
                 a0, b0, c0, e0, a1, b1, c1, e1, t_sh, *sems):
        c = lax.axis_index("c")
        s = lax.axis_index("s")
        wid = s * 2 + c
        slots = ((a0, b0, c0, e0), (a1, b1, c1, e1))
        gsem = sems[:_NSLOT]
        ssem = sems[_NSLOT:]

        pltpu.sync_copy(z_hbm.at[pl.ds(s * 640, 640), :],
                        t_sh.at[pl.ds(s * 640, 640), :])
        plsc.subcore_barrier()

        pltpu.sync_copy(src_hbm.at[:, wid], idx)

        def gathers(w, sl):
            A, B, C, E = slots[sl]
            pltpu.async_copy(g_hbm.at[idx.at[0, w]], A, gsem[sl])
            pltpu.async_copy(g_hbm.at[idx.at[1, w]], B, gsem[sl])
            pltpu.async_copy(g_hbm.at[idx.at[2, w]], C, gsem[sl])
            pltpu.async_copy(g_hbm.at[idx.at[4, w]], E, gsem[sl])

        def wait_gathers(w, sl):
            A, B, C, E = slots[sl]
            pltpu.make_async_copy(g_hbm.at[idx.at[0, w]], A, gsem[sl]).wait()
            pltpu.make_async_copy(g_hbm.at[idx.at[1, w]], B, gsem[sl]).wait()
            pltpu.make_async_copy(g_hbm.at[idx.at[2, w]], C, gsem[sl]).wait()
            pltpu.make_async_copy(g_hbm.at[idx.at[4, w]], E, gsem[sl]).wait()

        def scatters(w, sl):
            A, B, C, E = slots[sl]
            pltpu.async_copy(A, t_sh.at[idx.at[1, w]], ssem[sl], add=True)
            pltpu.async_copy(B, t_sh.at[idx.at[0, w]], ssem[sl], add=True)
            pltpu.async_copy(E, t_sh.at[idx.at[3, w]], ssem[sl], add=True)
            pltpu.async_copy(E, t_sh.at[idx.at[5, w]], ssem[sl], add=True)

        def wait_scatters(w, sl):
            A, B, C, E = slots[sl]
            pltpu.make_async_copy(A, t_sh.at[idx.at[1, w]], ssem[sl]).wait()
            pltpu.make_async_copy(B, t_sh.at[idx.at[0, w]], ssem[sl]).wait()
            pltpu.make_async_copy(E, t_sh.at[idx.at[3, w]], ssem[sl]).wait()
            pltpu.make_async_copy(E, t_sh.at[idx.at[5, w]], ssem[sl]).wait()

        def compute(sl):
            A, B, C, E = slots[sl]

            @pl.loop(0, WIN)
            def _row(e):
                for ch in range(nch):
                    cs = pl.ds(ch * 16, 16)
                    m = C[e, cs] + E[e, cs]
                    sab = A[e, cs] + B[e, cs]
                    C[e, cs] = m
                    E[e, cs] = sab
                    A[e, cs] = A[e, cs] + m
                    B[e, cs] = B[e, cs] + m

        gathers(0, 0)
        gathers(1, 1)

        @pl.loop(0, NWIN // _NSLOT)
        def _win(q):
            for sl in range(_NSLOT):
                w = q * _NSLOT + sl
                wait_gathers(w, sl)
                compute(sl)
                scatters(w, sl)
                wait_scatters(w, sl)

                @pl.when(w + _NSLOT < NWIN)
                def _():
                    gathers(w + _NSLOT, sl)

        plsc.subcore_barrier()
        pltpu.sync_copy(t_sh.at[pl.ds(s * 640, 640), :],
                        t_out.at[c, pl.ds(s * 640, 640), :])

    return _scatter


_scatter_d1 = _make_scatter(D1)
_scatter_d2 = _make_scatter(D2)


def kernel(H, hyperedges, rv, W1, b1, W2, b2):
    f32 = jnp.float32
    h_pad = jnp.zeros((VR, DIN), f32).at[0:V, :].set(H)
    padv = (V + (jnp.arange(NPAD, dtype=jnp.int32) % (VR - V)))[:, None]
    he_pad = jnp.concatenate(
        [hyperedges.astype(jnp.int32), jnp.broadcast_to(padv, (NPAD, K))], 0)
    hed2 = he_pad.T
    w2p = jnp.zeros((D1, D2), f32).at[:, 0:NCLS].set(W2)
    b2p = jnp.zeros((D2,), f32).at[0:NCLS].set(b2)
    z1 = jnp.zeros((VR,), f32)
    zd1 = jnp.zeros((VR, D1), f32)
    zd2 = jnp.zeros((VR, D2), f32)

    q, hw1 = _tc1(h_pad, W1, rv)
    src_all, deg_part = _sc_edges(hed2, q, z1)
    src4 = src_all.reshape(6, NW, NWIN, WIN)
    ds, dinv, g1 = _tc2(deg_part, hw1)
    t1 = _scatter_d1(g1, src4, zd1)
    hw2, g2 = _tc3(t1, hw1, ds, dinv, b1, w2p)
    t2 = _scatter_d2(g2, src4, zd2)
    return _tc4(t2, hw2, ds, dinv, b2p)
